# Initial kernel scaffold; baseline (speedup 1.0000x reference)
import jax
import jax.numpy as jnp
from jax.experimental import pallas as pl


def kernel(node_feat, edge_index, edge_attr, params):
    raise NotImplementedError("write your pallas kernel here")


# speedup vs baseline: 2.8141x; 2.8141x over previous
import functools

import jax
import jax.numpy as jnp
from jax import lax
from jax.experimental import pallas as pl
from jax.experimental.pallas import tpu as pltpu
from jax.experimental.pallas import tpu_sc as plsc

N = 10000
E = 320000
H = 128
L = 4
NFEAT = 9
EFEAT = 3
AVOCAB = 128
BVOCAB = 8
TARGET = 1

NPAD = 10240
BN = 1024
BE = 2000
NBLK = NPAD // BN
EBLK = E // BE

NW = 32
EPW = E // NW
C = 80
NCH = EPW // C
G = 5
GOUT = NCH // G
NROWS_T = NPAD // 16
ZR = 128

F32 = jnp.float32


def _silu(x):
    return x * jax.nn.sigmoid(x)


def _make_encoder(nrows, nfeat, ncols, block):
    def body(f_ref, emb_ref, out_ref):
        feats = f_ref[...]
        iota = lax.broadcasted_iota(jnp.int32, (block, AVOCAB), 1)
        acc = jnp.zeros((block, H), F32)
        for i in range(nfeat):
            oh = (feats[:, i][:, None] == iota).astype(F32)
            acc = acc + jnp.dot(oh, emb_ref[i], preferred_element_type=F32)
        out_ref[...] = acc

    return pl.pallas_call(
        body,
        grid=(nrows // block,),
        in_specs=[
            pl.BlockSpec((block, ncols), lambda i: (i, 0)),
            pl.BlockSpec((nfeat, AVOCAB, H), lambda i: (0, 0, 0)),
        ],
        out_specs=pl.BlockSpec((block, H), lambda i: (i, 0)),
        out_shape=jax.ShapeDtypeStruct((nrows, H), F32),
    )


_enc_node = _make_encoder(NPAD, NFEAT, 16, BN)
_enc_edge = _make_encoder(E, EFEAT, 8, BE)


def _ab_body(h_ref, w_ref, a_ref, b_ref):
    hb = h_ref[...]
    a_ref[...] = jnp.dot(hb, w_ref[0], preferred_element_type=F32)
    b_ref[...] = jnp.dot(hb, w_ref[1], preferred_element_type=F32)


_ab = pl.pallas_call(
    _ab_body,
    grid=(NBLK,),
    in_specs=[
        pl.BlockSpec((BN, H), lambda i: (i, 0)),
        pl.BlockSpec((2, H, H), lambda i: (0, 0, 0)),
    ],
    out_specs=[
        pl.BlockSpec((BN, H), lambda i: (i, 0)),
        pl.BlockSpec((BN, H), lambda i: (i, 0)),
    ],
    out_shape=[
        jax.ShapeDtypeStruct((NPAD, H), F32),
        jax.ShapeDtypeStruct((NPAD, H), F32),
    ],
)


def _gather_body(a_hbm, b_hbm, src_hbm, dst_hbm, ao_hbm, bo_hbm,
                 sidx, didx, abuf, bbuf, sema, semb):
    cid = lax.axis_index("c")
    sid = lax.axis_index("s")
    wid = sid * 2 + cid
    pltpu.sync_copy(src_hbm.at[pl.ds(wid * EPW, EPW)], sidx)
    pltpu.sync_copy(dst_hbm.at[pl.ds(wid * EPW, EPW)], didx)

    def outer(o, carry):
        for j in range(G):
            k = o * G + j
            ca = pltpu.async_copy(a_hbm.at[sidx.at[pl.ds(k * C, C)]],
                                  abuf.at[pl.ds(j * C, C)], sema)
            cb = pltpu.async_copy(b_hbm.at[didx.at[pl.ds(k * C, C)]],
                                  bbuf.at[pl.ds(j * C, C)], semb)
            ca.wait()
            cb.wait()
        base = wid * EPW + o * (G * C)
        pltpu.sync_copy(abuf, ao_hbm.at[pl.ds(base, G * C)])
        pltpu.sync_copy(bbuf, bo_hbm.at[pl.ds(base, G * C)])
        return carry

    lax.fori_loop(0, GOUT, outer, 0)


@functools.cache
def _build_gather():
    return functools.partial(
        pl.kernel,
        out_type=(
            jax.ShapeDtypeStruct((E, H), F32),
            jax.ShapeDtypeStruct((E, H), F32),
        ),
        mesh=plsc.VectorSubcoreMesh(core_axis_name="c", subcore_axis_name="s"),
        scratch_types=[
            pltpu.VMEM((EPW,), jnp.int32),
            pltpu.VMEM((EPW,), jnp.int32),
            pltpu.VMEM((G * C, H), F32),
            pltpu.VMEM((G * C, H), F32),
            pltpu.SemaphoreType.DMA,
            pltpu.SemaphoreType.DMA,
        ],
    )(_gather_body)


def _msg_body(a_ref, b_ref, e_ref, w1e_ref, b1_ref, w2_ref, b2_ref,
              swt_ref, sb_ref, m_ref):
    t = (a_ref[...] + b_ref[...]
         + jnp.dot(e_ref[...], w1e_ref[...], preferred_element_type=F32)
         + b1_ref[...])
    t = _silu(t)
    mm = _silu(jnp.dot(t, w2_ref[...], preferred_element_type=F32) + b2_ref[...])
    gate = jax.nn.sigmoid(
        jnp.sum(mm * swt_ref[...], axis=1, keepdims=True) + sb_ref[...])
    m_ref[...] = mm * gate


_msg = pl.pallas_call(
    _msg_body,
    grid=(EBLK,),
    in_specs=[
        pl.BlockSpec((BE, H), lambda i: (i, 0)),
        pl.BlockSpec((BE, H), lambda i: (i, 0)),
        pl.BlockSpec((BE, H), lambda i: (i, 0)),
        pl.BlockSpec((H, H), lambda i: (0, 0)),
        pl.BlockSpec((1, H), lambda i: (0, 0)),
        pl.BlockSpec((H, H), lambda i: (0, 0)),
        pl.BlockSpec((1, H), lambda i: (0, 0)),
        pl.BlockSpec((1, H), lambda i: (0, 0)),
        pl.BlockSpec((1, 1), lambda i: (0, 0)),
    ],
    out_specs=pl.BlockSpec((BE, H), lambda i: (i, 0)),
    out_shape=jax.ShapeDtypeStruct((E, H), F32),
)


def _scatter_body(m_hbm, dst_hbm, z_hbm, out_hbm, didx, mbuf, zbuf, shared):
    cid = lax.axis_index("c")
    sid = lax.axis_index("s")
    wid = sid * 2 + cid
    pltpu.sync_copy(z_hbm, zbuf)
    for k in range(NROWS_T // ZR):
        pltpu.sync_copy(zbuf, shared.at[pl.ds(sid * NROWS_T + k * ZR, ZR)])
    plsc.subcore_barrier()
    pltpu.sync_copy(dst_hbm.at[wid], didx)

    def chunk(k, carry):
        base = wid * EPW + k * C
        pltpu.sync_copy(m_hbm.at[pl.ds(base, C)], mbuf)
        pltpu.sync_copy(mbuf, shared.at[didx.at[k]], add=True)
        return carry

    lax.fori_loop(0, NCH, chunk, 0)
    plsc.subcore_barrier()
    for k in range(NROWS_T // ZR):
        r0 = sid * NROWS_T + k * ZR
        pltpu.sync_copy(shared.at[pl.ds(r0, ZR)], zbuf)
        pltpu.sync_copy(zbuf, out_hbm.at[cid, pl.ds(r0, ZR)])


@functools.cache
def _build_scatter():
    return functools.partial(
        pl.kernel,
        out_type=jax.ShapeDtypeStruct((2, NPAD, H), F32),
        mesh=plsc.VectorSubcoreMesh(core_axis_name="c", subcore_axis_name="s"),
        scratch_types=[
            pltpu.VMEM((NCH, C), jnp.int32),
            pltpu.VMEM((C, H), F32),
            pltpu.VMEM((ZR, H), F32),
            pltpu.VMEM_SHARED((NPAD, H), F32),
        ],
    )(_scatter_body)


def _make_upd(with_ab):
    def body(p_ref, h_ref, w_ref, b_ref, *rest):
        if with_ab:
            wab_ref, h_out, a_out, b_out = rest
        else:
            (h_out,) = rest
        hb = h_ref[...]
        u = p_ref[0] + p_ref[1] + hb
        hn = _silu(jnp.dot(u, w_ref[0], preferred_element_type=F32) + b_ref[0])
        hn = jnp.dot(hn, w_ref[1], preferred_element_type=F32) + b_ref[1]
        hnew = hn + hb
        h_out[...] = hnew
        if with_ab:
            a_out[...] = jnp.dot(hnew, wab_ref[0], preferred_element_type=F32)
            b_out[...] = jnp.dot(hnew, wab_ref[1], preferred_element_type=F32)

    in_specs = [
        pl.BlockSpec((2, BN, H), lambda i: (0, i, 0)),
        pl.BlockSpec((BN, H), lambda i: (i, 0)),
        pl.BlockSpec((2, H, H), lambda i: (0, 0, 0)),
        pl.BlockSpec((2, 1, H), lambda i: (0, 0, 0)),
    ]
    out_specs = [pl.BlockSpec((BN, H), lambda i: (i, 0))]
    out_shape = [jax.ShapeDtypeStruct((NPAD, H), F32)]
    if with_ab:
        in_specs.append(pl.BlockSpec((2, H, H), lambda i: (0, 0, 0)))
        out_specs += [pl.BlockSpec((BN, H), lambda i: (i, 0))] * 2
        out_shape += [jax.ShapeDtypeStruct((NPAD, H), F32)] * 2
    return pl.pallas_call(
        body, grid=(NBLK,), in_specs=in_specs,
        out_specs=out_specs, out_shape=out_shape,
    )


_upd_ab = _make_upd(True)
_upd = _make_upd(False)


def _read_body(h_ref, ow_ref, ob_ref, rw1_ref, rb1_ref, rw2t_ref, rb2_ref,
               out_ref, acc_ref):
    i = pl.program_id(0)

    @pl.when(i == 0)
    def _():
        acc_ref[...] = jnp.zeros_like(acc_ref)

    hb = h_ref[...]
    ho = _silu(jnp.dot(hb, ow_ref[0], preferred_element_type=F32) + ob_ref[0])
    ho = jnp.dot(ho, ow_ref[1], preferred_element_type=F32) + ob_ref[1]
    rows = lax.broadcasted_iota(jnp.int32, (BN, H), 0) + i * BN
    ho = jnp.where(rows < N, ho, 0.0)
    acc_ref[...] += jnp.sum(ho, axis=0, keepdims=True)

    @pl.when(i == NBLK - 1)
    def _():
        s = acc_ref[...]
        cat = jnp.concatenate([s, s / N], axis=1)
        r = _silu(jnp.dot(cat, rw1_ref[...], preferred_element_type=F32)
                  + rb1_ref[...])
        out_ref[...] = (jnp.sum(r * rw2t_ref[...], axis=1, keepdims=True)
                        + rb2_ref[...])


_readout = pl.pallas_call(
    _read_body,
    grid=(NBLK,),
    in_specs=[
        pl.BlockSpec((BN, H), lambda i: (i, 0)),
        pl.BlockSpec((2, H, H), lambda i: (0, 0, 0)),
        pl.BlockSpec((2, 1, H), lambda i: (0, 0, 0)),
        pl.BlockSpec((2 * H, H), lambda i: (0, 0)),
        pl.BlockSpec((1, H), lambda i: (0, 0)),
        pl.BlockSpec((1, H), lambda i: (0, 0)),
        pl.BlockSpec((1, 1), lambda i: (0, 0)),
    ],
    out_specs=pl.BlockSpec((1, 1), lambda i: (0, 0)),
    out_shape=jax.ShapeDtypeStruct((1, TARGET), F32),
    scratch_shapes=[pltpu.VMEM((1, H), F32)],
    compiler_params=pltpu.CompilerParams(
        dimension_semantics=("arbitrary",)),
)


def kernel(node_feat, edge_index, edge_attr, params):
    p = params
    nf = jnp.pad(node_feat, ((0, NPAD - N), (0, 16 - NFEAT)))
    ea = jnp.pad(edge_attr, ((0, 0), (0, 8 - EFEAT)))
    src = edge_index[0]
    dst = edge_index[1]
    dst3 = dst.reshape(NW, NCH, C)
    bond = jnp.pad(p["bond_emb"], ((0, 0), (0, AVOCAB - BVOCAB), (0, 0)))

    h = _enc_node(nf, p["atom_emb"])
    e = _enc_edge(ea, bond)

    msg_W1 = p["msg_W1"]
    wab = jnp.stack([msg_W1[0, :H], msg_W1[0, H:2 * H]])
    A, B = _ab(h, wab)
    zrows = jnp.zeros((ZR, H), F32)

    for l in range(L):
        asrc, bdst = _build_gather()(A, B, src, dst)
        m = _msg(asrc, bdst, e,
                 msg_W1[l, 2 * H:],
                 p["msg_b1"][l][None, :],
                 p["msg_W2"][l],
                 p["msg_b2"][l][None, :],
                 p["soft_W"][l].T,
                 p["soft_b"][l][None, :])
        partials = _build_scatter()(m, dst3, zrows)
        uw = jnp.stack([p["upd_W1"][l], p["upd_W2"][l]])
        ub = jnp.stack([p["upd_b1"][l], p["upd_b2"][l]])[:, None, :]
        if l < L - 1:
            wabn = jnp.stack([msg_W1[l + 1, :H], msg_W1[l + 1, H:2 * H]])
            h, A, B = _upd_ab(partials, h, uw, ub, wabn)
        else:
            (h,) = _upd(partials, h, uw, ub)

    ow = jnp.stack([p["on_W1"], p["on_W2"]])
    ob = jnp.stack([p["on_b1"], p["on_b2"]])[:, None, :]
    out = _readout(h, ow, ob,
                   p["ro_W1"],
                   p["ro_b1"][None, :],
                   p["ro_W2"].T,
                   p["ro_b2"][None, :])
    return out


# speedup vs baseline: 3.0794x; 1.0943x over previous
import functools

import jax
import jax.numpy as jnp
from jax import lax
from jax.experimental import pallas as pl
from jax.experimental.pallas import tpu as pltpu
from jax.experimental.pallas import tpu_sc as plsc

N = 10000
E = 320000
H = 128
L = 4
NFEAT = 9
EFEAT = 3
AVOCAB = 128
BVOCAB = 8
TARGET = 1

NPAD = 10240
BN = 1024
BE = 2000
NBLK = NPAD // BN
EBLK = E // BE

NW = 32
EPW = E // NW
C = 80
NCH = EPW // C
GC = 40
GG = 5
SLOT = GC * GG
GOUT = EPW // SLOT
NROWS_T = NPAD // 16
ZR = 64
NZC = NROWS_T // ZR

F32 = jnp.float32


def _silu(x):
    return x * jax.nn.sigmoid(x)


def _make_encoder(nrows, nfeat, ncols, block):
    def body(f_ref, emb_ref, out_ref):
        feats = f_ref[...]
        iota = lax.broadcasted_iota(jnp.int32, (block, AVOCAB), 1)
        acc = jnp.zeros((block, H), F32)
        for i in range(nfeat):
            oh = (feats[:, i][:, None] == iota).astype(F32)
            acc = acc + jnp.dot(oh, emb_ref[i], preferred_element_type=F32)
        out_ref[...] = acc

    return pl.pallas_call(
        body,
        grid=(nrows // block,),
        in_specs=[
            pl.BlockSpec((block, ncols), lambda i: (i, 0)),
            pl.BlockSpec((nfeat, AVOCAB, H), lambda i: (0, 0, 0)),
        ],
        out_specs=pl.BlockSpec((block, H), lambda i: (i, 0)),
        out_shape=jax.ShapeDtypeStruct((nrows, H), F32),
    )


_enc_node = _make_encoder(NPAD, NFEAT, 16, BN)
_enc_edge = _make_encoder(E, EFEAT, 8, BE)


def _ab_body(h_ref, w_ref, a_ref, b_ref):
    hb = h_ref[...]
    a_ref[...] = jnp.dot(hb, w_ref[0], preferred_element_type=F32)
    b_ref[...] = jnp.dot(hb, w_ref[1], preferred_element_type=F32)


_ab = pl.pallas_call(
    _ab_body,
    grid=(NBLK,),
    in_specs=[
        pl.BlockSpec((BN, H), lambda i: (i, 0)),
        pl.BlockSpec((2, H, H), lambda i: (0, 0, 0)),
    ],
    out_specs=[
        pl.BlockSpec((BN, H), lambda i: (i, 0)),
        pl.BlockSpec((BN, H), lambda i: (i, 0)),
    ],
    out_shape=[
        jax.ShapeDtypeStruct((NPAD, H), F32),
        jax.ShapeDtypeStruct((NPAD, H), F32),
    ],
)


def _gather_body(a_hbm, b_hbm, src_hbm, dst_hbm, ao_hbm, bo_hbm,
                 sidx, didx, ab0, bb0, ab1, bb1, sema, semw0, semw1):
    cid = lax.axis_index("c")
    sid = lax.axis_index("s")
    wid = sid * 2 + cid
    pltpu.sync_copy(src_hbm.at[pl.ds(wid * EPW, EPW)], sidx)
    pltpu.sync_copy(dst_hbm.at[pl.ds(wid * EPW, EPW)], didx)

    def issue_gathers(o, ab, bb):
        for j in range(GG):
            pltpu.async_copy(a_hbm.at[sidx.at[pl.ds((o * GG + j) * GC, GC)]],
                             ab.at[pl.ds(j * GC, GC)], sema)
            pltpu.async_copy(b_hbm.at[didx.at[pl.ds((o * GG + j) * GC, GC)]],
                             bb.at[pl.ds(j * GC, GC)], sema)

    def drain_gathers(ab, bb):
        pltpu.make_async_copy(a_hbm.at[pl.ds(0, SLOT)], ab, sema).wait()
        pltpu.make_async_copy(b_hbm.at[pl.ds(0, SLOT)], bb, sema).wait()

    def issue_writes(o, ab, bb, semw):
        base = wid * EPW + o * SLOT
        pltpu.async_copy(ab, ao_hbm.at[pl.ds(base, SLOT)], semw)
        pltpu.async_copy(bb, bo_hbm.at[pl.ds(base, SLOT)], semw)

    def drain_writes(ab, bb, semw):
        pltpu.make_async_copy(ab, ao_hbm.at[pl.ds(0, SLOT)], semw).wait()
        pltpu.make_async_copy(bb, bo_hbm.at[pl.ds(0, SLOT)], semw).wait()

    issue_gathers(0, ab0, bb0)
    drain_gathers(ab0, bb0)
    issue_writes(0, ab0, bb0, semw0)
    issue_gathers(1, ab1, bb1)

    def pair(t, carry):
        o1 = 1 + 2 * t
        drain_gathers(ab1, bb1)
        issue_writes(o1, ab1, bb1, semw1)
        drain_writes(ab0, bb0, semw0)
        issue_gathers(o1 + 1, ab0, bb0)
        drain_gathers(ab0, bb0)
        issue_writes(o1 + 1, ab0, bb0, semw0)
        drain_writes(ab1, bb1, semw1)
        issue_gathers(o1 + 2, ab1, bb1)
        return carry

    lax.fori_loop(0, (GOUT - 2) // 2, pair, 0)
    drain_gathers(ab1, bb1)
    issue_writes(GOUT - 1, ab1, bb1, semw1)
    drain_writes(ab0, bb0, semw0)
    drain_writes(ab1, bb1, semw1)


@functools.cache
def _build_gather():
    return functools.partial(
        pl.kernel,
        out_type=(
            jax.ShapeDtypeStruct((E, H), F32),
            jax.ShapeDtypeStruct((E, H), F32),
        ),
        mesh=plsc.VectorSubcoreMesh(core_axis_name="c", subcore_axis_name="s"),
        scratch_types=[
            pltpu.VMEM((EPW,), jnp.int32),
            pltpu.VMEM((EPW,), jnp.int32),
            pltpu.VMEM((SLOT, H), F32),
            pltpu.VMEM((SLOT, H), F32),
            pltpu.VMEM((SLOT, H), F32),
            pltpu.VMEM((SLOT, H), F32),
            pltpu.SemaphoreType.DMA,
            pltpu.SemaphoreType.DMA,
            pltpu.SemaphoreType.DMA,
        ],
    )(_gather_body)


def _msg_body(a_ref, b_ref, e_ref, w1e_ref, b1_ref, w2_ref, b2_ref,
              swt_ref, sb_ref, m_ref):
    t = (a_ref[...] + b_ref[...]
         + jnp.dot(e_ref[...], w1e_ref[...], preferred_element_type=F32)
         + b1_ref[...])
    t = _silu(t)
    mm = _silu(jnp.dot(t, w2_ref[...], preferred_element_type=F32) + b2_ref[...])
    gate = jax.nn.sigmoid(
        jnp.sum(mm * swt_ref[...], axis=1, keepdims=True) + sb_ref[...])
    m_ref[...] = mm * gate


_msg = pl.pallas_call(
    _msg_body,
    grid=(EBLK,),
    in_specs=[
        pl.BlockSpec((BE, H), lambda i: (i, 0)),
        pl.BlockSpec((BE, H), lambda i: (i, 0)),
        pl.BlockSpec((BE, H), lambda i: (i, 0)),
        pl.BlockSpec((H, H), lambda i: (0, 0)),
        pl.BlockSpec((1, H), lambda i: (0, 0)),
        pl.BlockSpec((H, H), lambda i: (0, 0)),
        pl.BlockSpec((1, H), lambda i: (0, 0)),
        pl.BlockSpec((1, H), lambda i: (0, 0)),
        pl.BlockSpec((1, 1), lambda i: (0, 0)),
    ],
    out_specs=pl.BlockSpec((BE, H), lambda i: (i, 0)),
    out_shape=jax.ShapeDtypeStruct((E, H), F32),
)


def _scatter_body(m_hbm, dst_hbm, z_hbm, out_hbm,
                  didx, mb0, mb1, zbuf, shared, semz, semm0, semm1):
    cid = lax.axis_index("c")
    sid = lax.axis_index("s")
    wid = sid * 2 + cid
    pltpu.sync_copy(z_hbm, zbuf)
    for k in range(NZC):
        pltpu.async_copy(zbuf, shared.at[pl.ds(sid * NROWS_T + k * ZR, ZR)],
                         semz)
    pltpu.sync_copy(dst_hbm.at[wid], didx)
    for k in range(NZC):
        pltpu.make_async_copy(zbuf, shared.at[pl.ds(0, ZR)], semz).wait()
    plsc.subcore_barrier()

    ebase = wid * EPW

    def load(k, mb, semm):
        pltpu.async_copy(m_hbm.at[pl.ds(ebase + k * C, C)], mb, semm)

    def drain(mb, semm):
        pltpu.make_async_copy(m_hbm.at[pl.ds(0, C)], mb, semm).wait()

    load(0, mb0, semm0)

    def pair(t, carry):
        k0 = 2 * t
        drain(mb0, semm0)
        load(k0 + 1, mb1, semm1)
        pltpu.sync_copy(mb0, shared.at[didx.at[k0]], add=True)
        drain(mb1, semm1)
        load(k0 + 2, mb0, semm0)
        pltpu.sync_copy(mb1, shared.at[didx.at[k0 + 1]], add=True)
        return carry

    lax.fori_loop(0, (NCH - 1) // 2, pair, 0)
    drain(mb0, semm0)
    pltpu.sync_copy(mb0, shared.at[didx.at[NCH - 1]], add=True)
    plsc.subcore_barrier()
    for k in range(NZC):
        r0 = sid * NROWS_T + k * ZR
        pltpu.sync_copy(shared.at[pl.ds(r0, ZR)], zbuf)
        pltpu.sync_copy(zbuf, out_hbm.at[cid, pl.ds(r0, ZR)])


@functools.cache
def _build_scatter():
    return functools.partial(
        pl.kernel,
        out_type=jax.ShapeDtypeStruct((2, NPAD, H), F32),
        mesh=plsc.VectorSubcoreMesh(core_axis_name="c", subcore_axis_name="s"),
        scratch_types=[
            pltpu.VMEM((NCH, C), jnp.int32),
            pltpu.VMEM((C, H), F32),
            pltpu.VMEM((C, H), F32),
            pltpu.VMEM((ZR, H), F32),
            pltpu.VMEM_SHARED((NPAD, H), F32),
            pltpu.SemaphoreType.DMA,
            pltpu.SemaphoreType.DMA,
            pltpu.SemaphoreType.DMA,
        ],
    )(_scatter_body)


def _make_upd(with_ab):
    def body(p_ref, h_ref, w_ref, b_ref, *rest):
        if with_ab:
            wab_ref, h_out, a_out, b_out = rest
        else:
            (h_out,) = rest
        hb = h_ref[...]
        u = p_ref[0] + p_ref[1] + hb
        hn = _silu(jnp.dot(u, w_ref[0], preferred_element_type=F32) + b_ref[0])
        hn = jnp.dot(hn, w_ref[1], preferred_element_type=F32) + b_ref[1]
        hnew = hn + hb
        h_out[...] = hnew
        if with_ab:
            a_out[...] = jnp.dot(hnew, wab_ref[0], preferred_element_type=F32)
            b_out[...] = jnp.dot(hnew, wab_ref[1], preferred_element_type=F32)

    in_specs = [
        pl.BlockSpec((2, BN, H), lambda i: (0, i, 0)),
        pl.BlockSpec((BN, H), lambda i: (i, 0)),
        pl.BlockSpec((2, H, H), lambda i: (0, 0, 0)),
        pl.BlockSpec((2, 1, H), lambda i: (0, 0, 0)),
    ]
    out_specs = [pl.BlockSpec((BN, H), lambda i: (i, 0))]
    out_shape = [jax.ShapeDtypeStruct((NPAD, H), F32)]
    if with_ab:
        in_specs.append(pl.BlockSpec((2, H, H), lambda i: (0, 0, 0)))
        out_specs += [pl.BlockSpec((BN, H), lambda i: (i, 0))] * 2
        out_shape += [jax.ShapeDtypeStruct((NPAD, H), F32)] * 2
    return pl.pallas_call(
        body, grid=(NBLK,), in_specs=in_specs,
        out_specs=out_specs, out_shape=out_shape,
    )


_upd_ab = _make_upd(True)
_upd = _make_upd(False)


def _read_body(h_ref, ow_ref, ob_ref, rw1_ref, rb1_ref, rw2t_ref, rb2_ref,
               out_ref, acc_ref):
    i = pl.program_id(0)

    @pl.when(i == 0)
    def _():
        acc_ref[...] = jnp.zeros_like(acc_ref)

    hb = h_ref[...]
    ho = _silu(jnp.dot(hb, ow_ref[0], preferred_element_type=F32) + ob_ref[0])
    ho = jnp.dot(ho, ow_ref[1], preferred_element_type=F32) + ob_ref[1]
    rows = lax.broadcasted_iota(jnp.int32, (BN, H), 0) + i * BN
    ho = jnp.where(rows < N, ho, 0.0)
    acc_ref[...] += jnp.sum(ho, axis=0, keepdims=True)

    @pl.when(i == NBLK - 1)
    def _():
        s = acc_ref[...]
        cat = jnp.concatenate([s, s / N], axis=1)
        r = _silu(jnp.dot(cat, rw1_ref[...], preferred_element_type=F32)
                  + rb1_ref[...])
        out_ref[...] = (jnp.sum(r * rw2t_ref[...], axis=1, keepdims=True)
                        + rb2_ref[...])


_readout = pl.pallas_call(
    _read_body,
    grid=(NBLK,),
    in_specs=[
        pl.BlockSpec((BN, H), lambda i: (i, 0)),
        pl.BlockSpec((2, H, H), lambda i: (0, 0, 0)),
        pl.BlockSpec((2, 1, H), lambda i: (0, 0, 0)),
        pl.BlockSpec((2 * H, H), lambda i: (0, 0)),
        pl.BlockSpec((1, H), lambda i: (0, 0)),
        pl.BlockSpec((1, H), lambda i: (0, 0)),
        pl.BlockSpec((1, 1), lambda i: (0, 0)),
    ],
    out_specs=pl.BlockSpec((1, 1), lambda i: (0, 0)),
    out_shape=jax.ShapeDtypeStruct((1, TARGET), F32),
    scratch_shapes=[pltpu.VMEM((1, H), F32)],
    compiler_params=pltpu.CompilerParams(
        dimension_semantics=("arbitrary",)),
)


def kernel(node_feat, edge_index, edge_attr, params):
    p = params
    nf = jnp.pad(node_feat, ((0, NPAD - N), (0, 16 - NFEAT)))
    ea = jnp.pad(edge_attr, ((0, 0), (0, 8 - EFEAT)))
    src = edge_index[0]
    dst = edge_index[1]
    dst3 = dst.reshape(NW, NCH, C)
    bond = jnp.pad(p["bond_emb"], ((0, 0), (0, AVOCAB - BVOCAB), (0, 0)))

    h = _enc_node(nf, p["atom_emb"])
    e = _enc_edge(ea, bond)

    msg_W1 = p["msg_W1"]
    wab = jnp.stack([msg_W1[0, :H], msg_W1[0, H:2 * H]])
    A, B = _ab(h, wab)
    zrows = jnp.zeros((ZR, H), F32)

    for l in range(L):
        asrc, bdst = _build_gather()(A, B, src, dst)
        m = _msg(asrc, bdst, e,
                 msg_W1[l, 2 * H:],
                 p["msg_b1"][l][None, :],
                 p["msg_W2"][l],
                 p["msg_b2"][l][None, :],
                 p["soft_W"][l].T,
                 p["soft_b"][l][None, :])
        partials = _build_scatter()(m, dst3, zrows)
        uw = jnp.stack([p["upd_W1"][l], p["upd_W2"][l]])
        ub = jnp.stack([p["upd_b1"][l], p["upd_b2"][l]])[:, None, :]
        if l < L - 1:
            wabn = jnp.stack([msg_W1[l + 1, :H], msg_W1[l + 1, H:2 * H]])
            h, A, B = _upd_ab(partials, h, uw, ub, wabn)
        else:
            (h,) = _upd(partials, h, uw, ub)

    ow = jnp.stack([p["on_W1"], p["on_W2"]])
    ob = jnp.stack([p["on_b1"], p["on_b2"]])[:, None, :]
    out = _readout(h, ow, ob,
                   p["ro_W1"],
                   p["ro_b1"][None, :],
                   p["ro_W2"].T,
                   p["ro_b2"][None, :])
    return out


# speedup vs baseline: 3.2772x; 1.0642x over previous
import functools

import jax
import jax.numpy as jnp
from jax import lax
from jax.experimental import pallas as pl
from jax.experimental.pallas import tpu as pltpu
from jax.experimental.pallas import tpu_sc as plsc

N = 10000
E = 320000
H = 128
L = 4
NFEAT = 9
EFEAT = 3
AVOCAB = 128
BVOCAB = 8
TARGET = 1

NPAD = 10240
BN = 1024
BE = 2000
NBLK = NPAD // BN
EBLK = E // BE

NW = 32
EPW = E // NW
C = 80
NCH = EPW // C
GC = 40
GG = 5
SLOT = GC * GG
GOUT = EPW // SLOT
NROWS_T = NPAD // 16
ZR = 64
NZC = NROWS_T // ZR

F32 = jnp.float32


def _silu(x):
    return x * jax.nn.sigmoid(x)


def _make_encoder(nrows, nfeat, ncols, block):
    def body(f_ref, emb_ref, out_ref):
        feats = f_ref[...]
        iota = lax.broadcasted_iota(jnp.int32, (block, AVOCAB), 1)
        acc = jnp.zeros((block, H), F32)
        for i in range(nfeat):
            oh = (feats[:, i][:, None] == iota).astype(F32)
            acc = acc + jnp.dot(oh, emb_ref[i], preferred_element_type=F32)
        out_ref[...] = acc

    return pl.pallas_call(
        body,
        grid=(nrows // block,),
        in_specs=[
            pl.BlockSpec((block, ncols), lambda i: (i, 0)),
            pl.BlockSpec((nfeat, AVOCAB, H), lambda i: (0, 0, 0)),
        ],
        out_specs=pl.BlockSpec((block, H), lambda i: (i, 0)),
        out_shape=jax.ShapeDtypeStruct((nrows, H), F32),
    )


_enc_node = _make_encoder(NPAD, NFEAT, 16, BN)
_enc_edge = _make_encoder(E, EFEAT, 8, BE)


def _ab_body(h_ref, w_ref, a_ref, b_ref):
    hb = h_ref[...]
    a_ref[...] = jnp.dot(hb, w_ref[0], preferred_element_type=F32)
    b_ref[...] = jnp.dot(hb, w_ref[1], preferred_element_type=F32)


_ab = pl.pallas_call(
    _ab_body,
    grid=(NBLK,),
    in_specs=[
        pl.BlockSpec((BN, H), lambda i: (i, 0)),
        pl.BlockSpec((2, H, H), lambda i: (0, 0, 0)),
    ],
    out_specs=[
        pl.BlockSpec((BN, H), lambda i: (i, 0)),
        pl.BlockSpec((BN, H), lambda i: (i, 0)),
    ],
    out_shape=[
        jax.ShapeDtypeStruct((NPAD, H), F32),
        jax.ShapeDtypeStruct((NPAD, H), F32),
    ],
)


def _add_into(ab, bb):
    def body(r):
        for c_ in range(H // 16):
            sl = (r, pl.ds(c_ * 16, 16))
            ab[sl] = ab[sl] + bb[sl]

    plsc.parallel_loop(0, SLOT, 1, unroll=4)(body)


def _gather_body(a_hbm, b_hbm, src_hbm, dst_hbm, so_hbm,
                 sidx, didx, ab0, bb0, ab1, bb1, sema, semw0, semw1):
    cid = lax.axis_index("c")
    sid = lax.axis_index("s")
    wid = sid * 2 + cid
    pltpu.sync_copy(src_hbm.at[pl.ds(wid * EPW, EPW)], sidx)
    pltpu.sync_copy(dst_hbm.at[pl.ds(wid * EPW, EPW)], didx)

    def issue_gathers(o, ab, bb):
        for j in range(GG):
            pltpu.async_copy(a_hbm.at[sidx.at[pl.ds((o * GG + j) * GC, GC)]],
                             ab.at[pl.ds(j * GC, GC)], sema)
            pltpu.async_copy(b_hbm.at[didx.at[pl.ds((o * GG + j) * GC, GC)]],
                             bb.at[pl.ds(j * GC, GC)], sema)

    def drain_gathers(ab, bb):
        pltpu.make_async_copy(a_hbm.at[pl.ds(0, SLOT)], ab, sema).wait()
        pltpu.make_async_copy(b_hbm.at[pl.ds(0, SLOT)], bb, sema).wait()

    def issue_write(o, ab, semw):
        base = wid * EPW + o * SLOT
        pltpu.async_copy(ab, so_hbm.at[pl.ds(base, SLOT)], semw)

    def drain_write(ab, semw):
        pltpu.make_async_copy(ab, so_hbm.at[pl.ds(0, SLOT)], semw).wait()

    issue_gathers(0, ab0, bb0)
    drain_gathers(ab0, bb0)
    issue_gathers(1, ab1, bb1)
    _add_into(ab0, bb0)
    issue_write(0, ab0, semw0)

    def pair(t, carry):
        o1 = 1 + 2 * t
        drain_gathers(ab1, bb1)
        drain_write(ab0, semw0)
        issue_gathers(o1 + 1, ab0, bb0)
        _add_into(ab1, bb1)
        issue_write(o1, ab1, semw1)
        drain_gathers(ab0, bb0)
        drain_write(ab1, semw1)
        issue_gathers(o1 + 2, ab1, bb1)
        _add_into(ab0, bb0)
        issue_write(o1 + 1, ab0, semw0)
        return carry

    lax.fori_loop(0, (GOUT - 2) // 2, pair, 0)
    drain_gathers(ab1, bb1)
    _add_into(ab1, bb1)
    issue_write(GOUT - 1, ab1, semw1)
    drain_write(ab0, semw0)
    drain_write(ab1, semw1)


@functools.cache
def _build_gather():
    return functools.partial(
        pl.kernel,
        out_type=jax.ShapeDtypeStruct((E, H), F32),
        mesh=plsc.VectorSubcoreMesh(core_axis_name="c", subcore_axis_name="s"),
        scratch_types=[
            pltpu.VMEM((EPW,), jnp.int32),
            pltpu.VMEM((EPW,), jnp.int32),
            pltpu.VMEM((SLOT, H), F32),
            pltpu.VMEM((SLOT, H), F32),
            pltpu.VMEM((SLOT, H), F32),
            pltpu.VMEM((SLOT, H), F32),
            pltpu.SemaphoreType.DMA,
            pltpu.SemaphoreType.DMA,
            pltpu.SemaphoreType.DMA,
        ],
    )(_gather_body)


def _msg_body(s_ref, e_ref, w1e_ref, b1_ref, w2_ref, b2_ref,
              swt_ref, sb_ref, m_ref):
    t = (s_ref[...]
         + jnp.dot(e_ref[...], w1e_ref[...], preferred_element_type=F32)
         + b1_ref[...])
    t = _silu(t)
    mm = _silu(jnp.dot(t, w2_ref[...], preferred_element_type=F32) + b2_ref[...])
    gate = jax.nn.sigmoid(
        jnp.sum(mm * swt_ref[...], axis=1, keepdims=True) + sb_ref[...])
    m_ref[...] = mm * gate


_msg = pl.pallas_call(
    _msg_body,
    grid=(EBLK,),
    in_specs=[
        pl.BlockSpec((BE, H), lambda i: (i, 0)),
        pl.BlockSpec((BE, H), lambda i: (i, 0)),
        pl.BlockSpec((H, H), lambda i: (0, 0)),
        pl.BlockSpec((1, H), lambda i: (0, 0)),
        pl.BlockSpec((H, H), lambda i: (0, 0)),
        pl.BlockSpec((1, H), lambda i: (0, 0)),
        pl.BlockSpec((1, H), lambda i: (0, 0)),
        pl.BlockSpec((1, 1), lambda i: (0, 0)),
    ],
    out_specs=pl.BlockSpec((BE, H), lambda i: (i, 0)),
    out_shape=jax.ShapeDtypeStruct((E, H), F32),
)


def _scatter_body(m_hbm, dst_hbm, z_hbm, out_hbm,
                  didx, mb0, mb1, zbuf, shared, semz, semm0, semm1):
    cid = lax.axis_index("c")
    sid = lax.axis_index("s")
    wid = sid * 2 + cid
    pltpu.sync_copy(z_hbm, zbuf)
    for k in range(NZC):
        pltpu.async_copy(zbuf, shared.at[pl.ds(sid * NROWS_T + k * ZR, ZR)],
                         semz)
    pltpu.sync_copy(dst_hbm.at[wid], didx)
    for k in range(NZC):
        pltpu.make_async_copy(zbuf, shared.at[pl.ds(0, ZR)], semz).wait()
    plsc.subcore_barrier()

    ebase = wid * EPW

    def load(k, mb, semm):
        pltpu.async_copy(m_hbm.at[pl.ds(ebase + k * C, C)], mb, semm)

    def drain(mb, semm):
        pltpu.make_async_copy(m_hbm.at[pl.ds(0, C)], mb, semm).wait()

    load(0, mb0, semm0)

    def pair(t, carry):
        k0 = 2 * t
        drain(mb0, semm0)
        load(k0 + 1, mb1, semm1)
        pltpu.sync_copy(mb0, shared.at[didx.at[k0]], add=True)
        drain(mb1, semm1)
        load(k0 + 2, mb0, semm0)
        pltpu.sync_copy(mb1, shared.at[didx.at[k0 + 1]], add=True)
        return carry

    lax.fori_loop(0, (NCH - 1) // 2, pair, 0)
    drain(mb0, semm0)
    pltpu.sync_copy(mb0, shared.at[didx.at[NCH - 1]], add=True)
    plsc.subcore_barrier()
    for k in range(NZC):
        r0 = sid * NROWS_T + k * ZR
        pltpu.sync_copy(shared.at[pl.ds(r0, ZR)], zbuf)
        pltpu.sync_copy(zbuf, out_hbm.at[cid, pl.ds(r0, ZR)])


@functools.cache
def _build_scatter():
    return functools.partial(
        pl.kernel,
        out_type=jax.ShapeDtypeStruct((2, NPAD, H), F32),
        mesh=plsc.VectorSubcoreMesh(core_axis_name="c", subcore_axis_name="s"),
        scratch_types=[
            pltpu.VMEM((NCH, C), jnp.int32),
            pltpu.VMEM((C, H), F32),
            pltpu.VMEM((C, H), F32),
            pltpu.VMEM((ZR, H), F32),
            pltpu.VMEM_SHARED((NPAD, H), F32),
            pltpu.SemaphoreType.DMA,
            pltpu.SemaphoreType.DMA,
            pltpu.SemaphoreType.DMA,
        ],
    )(_scatter_body)


def _make_upd(with_ab):
    def body(p_ref, h_ref, w_ref, b_ref, *rest):
        if with_ab:
            wab_ref, h_out, a_out, b_out = rest
        else:
            (h_out,) = rest
        hb = h_ref[...]
        u = p_ref[0] + p_ref[1] + hb
        hn = _silu(jnp.dot(u, w_ref[0], preferred_element_type=F32) + b_ref[0])
        hn = jnp.dot(hn, w_ref[1], preferred_element_type=F32) + b_ref[1]
        hnew = hn + hb
        h_out[...] = hnew
        if with_ab:
            a_out[...] = jnp.dot(hnew, wab_ref[0], preferred_element_type=F32)
            b_out[...] = jnp.dot(hnew, wab_ref[1], preferred_element_type=F32)

    in_specs = [
        pl.BlockSpec((2, BN, H), lambda i: (0, i, 0)),
        pl.BlockSpec((BN, H), lambda i: (i, 0)),
        pl.BlockSpec((2, H, H), lambda i: (0, 0, 0)),
        pl.BlockSpec((2, 1, H), lambda i: (0, 0, 0)),
    ]
    out_specs = [pl.BlockSpec((BN, H), lambda i: (i, 0))]
    out_shape = [jax.ShapeDtypeStruct((NPAD, H), F32)]
    if with_ab:
        in_specs.append(pl.BlockSpec((2, H, H), lambda i: (0, 0, 0)))
        out_specs += [pl.BlockSpec((BN, H), lambda i: (i, 0))] * 2
        out_shape += [jax.ShapeDtypeStruct((NPAD, H), F32)] * 2
    return pl.pallas_call(
        body, grid=(NBLK,), in_specs=in_specs,
        out_specs=out_specs, out_shape=out_shape,
    )


_upd_ab = _make_upd(True)
_upd = _make_upd(False)


def _read_body(h_ref, ow_ref, ob_ref, rw1_ref, rb1_ref, rw2t_ref, rb2_ref,
               out_ref, acc_ref):
    i = pl.program_id(0)

    @pl.when(i == 0)
    def _():
        acc_ref[...] = jnp.zeros_like(acc_ref)

    hb = h_ref[...]
    ho = _silu(jnp.dot(hb, ow_ref[0], preferred_element_type=F32) + ob_ref[0])
    ho = jnp.dot(ho, ow_ref[1], preferred_element_type=F32) + ob_ref[1]
    rows = lax.broadcasted_iota(jnp.int32, (BN, H), 0) + i * BN
    ho = jnp.where(rows < N, ho, 0.0)
    acc_ref[...] += jnp.sum(ho, axis=0, keepdims=True)

    @pl.when(i == NBLK - 1)
    def _():
        s = acc_ref[...]
        cat = jnp.concatenate([s, s / N], axis=1)
        r = _silu(jnp.dot(cat, rw1_ref[...], preferred_element_type=F32)
                  + rb1_ref[...])
        out_ref[...] = (jnp.sum(r * rw2t_ref[...], axis=1, keepdims=True)
                        + rb2_ref[...])


_readout = pl.pallas_call(
    _read_body,
    grid=(NBLK,),
    in_specs=[
        pl.BlockSpec((BN, H), lambda i: (i, 0)),
        pl.BlockSpec((2, H, H), lambda i: (0, 0, 0)),
        pl.BlockSpec((2, 1, H), lambda i: (0, 0, 0)),
        pl.BlockSpec((2 * H, H), lambda i: (0, 0)),
        pl.BlockSpec((1, H), lambda i: (0, 0)),
        pl.BlockSpec((1, H), lambda i: (0, 0)),
        pl.BlockSpec((1, 1), lambda i: (0, 0)),
    ],
    out_specs=pl.BlockSpec((1, 1), lambda i: (0, 0)),
    out_shape=jax.ShapeDtypeStruct((1, TARGET), F32),
    scratch_shapes=[pltpu.VMEM((1, H), F32)],
    compiler_params=pltpu.CompilerParams(
        dimension_semantics=("arbitrary",)),
)


def kernel(node_feat, edge_index, edge_attr, params):
    p = params
    nf = jnp.pad(node_feat, ((0, NPAD - N), (0, 16 - NFEAT)))
    ea = jnp.pad(edge_attr, ((0, 0), (0, 8 - EFEAT)))
    src = edge_index[0]
    dst = edge_index[1]
    dst3 = dst.reshape(NW, NCH, C)
    bond = jnp.pad(p["bond_emb"], ((0, 0), (0, AVOCAB - BVOCAB), (0, 0)))

    h = _enc_node(nf, p["atom_emb"])
    e = _enc_edge(ea, bond)

    msg_W1 = p["msg_W1"]
    wab = jnp.stack([msg_W1[0, :H], msg_W1[0, H:2 * H]])
    A, B = _ab(h, wab)
    zrows = jnp.zeros((ZR, H), F32)

    for l in range(L):
        s_sum = _build_gather()(A, B, src, dst)
        m = _msg(s_sum, e,
                 msg_W1[l, 2 * H:],
                 p["msg_b1"][l][None, :],
                 p["msg_W2"][l],
                 p["msg_b2"][l][None, :],
                 p["soft_W"][l].T,
                 p["soft_b"][l][None, :])
        partials = _build_scatter()(m, dst3, zrows)
        uw = jnp.stack([p["upd_W1"][l], p["upd_W2"][l]])
        ub = jnp.stack([p["upd_b1"][l], p["upd_b2"][l]])[:, None, :]
        if l < L - 1:
            wabn = jnp.stack([msg_W1[l + 1, :H], msg_W1[l + 1, H:2 * H]])
            h, A, B = _upd_ab(partials, h, uw, ub, wabn)
        else:
            (h,) = _upd(partials, h, uw, ub)

    ow = jnp.stack([p["on_W1"], p["on_W2"]])
    ob = jnp.stack([p["on_b1"], p["on_b2"]])[:, None, :]
    out = _readout(h, ow, ob,
                   p["ro_W1"],
                   p["ro_b1"][None, :],
                   p["ro_W2"].T,
                   p["ro_b2"][None, :])
    return out


# speedup vs baseline: 3.5807x; 1.0926x over previous
import functools

import jax
import jax.numpy as jnp
from jax import lax
from jax.experimental import pallas as pl
from jax.experimental.pallas import tpu as pltpu
from jax.experimental.pallas import tpu_sc as plsc

N = 10000
E = 320000
H = 128
L = 4
NFEAT = 9
EFEAT = 3
AVOCAB = 128
BVOCAB = 8
TARGET = 1

NPAD = 10240
BN = 1024
BE = 2000
NBLK = NPAD // BN
EBLK = E // BE

NW = 32
EH = E // 2
GC = 40
GG = 5
SLOT = GC * GG
SCC = 40
NROWS_T = NPAD // 16
ZR = 64
NZC = NROWS_T // ZR

F32 = jnp.float32


def _silu(x):
    return x * jax.nn.sigmoid(x)


def _make_encoder(nrows, nfeat, ncols, block):
    def body(f_ref, emb_ref, out_ref):
        feats = f_ref[...]
        iota = lax.broadcasted_iota(jnp.int32, (block, AVOCAB), 1)
        acc = jnp.zeros((block, H), F32)
        for i in range(nfeat):
            oh = (feats[:, i][:, None] == iota).astype(F32)
            acc = acc + jnp.dot(oh, emb_ref[i], preferred_element_type=F32)
        out_ref[...] = acc

    return pl.pallas_call(
        body,
        grid=(nrows // block,),
        in_specs=[
            pl.BlockSpec((block, ncols), lambda i: (i, 0)),
            pl.BlockSpec((nfeat, AVOCAB, H), lambda i: (0, 0, 0)),
        ],
        out_specs=pl.BlockSpec((block, H), lambda i: (i, 0)),
        out_shape=jax.ShapeDtypeStruct((nrows, H), F32),
    )


_enc_node = _make_encoder(NPAD, NFEAT, 16, BN)
_enc_edge = _make_encoder(EH, EFEAT, 8, BE)


def _ab_body(h_ref, w_ref, a_ref, b_ref):
    hb = h_ref[...]
    a_ref[...] = jnp.dot(hb, w_ref[0], preferred_element_type=F32)
    b_ref[...] = jnp.dot(hb, w_ref[1], preferred_element_type=F32)


_ab = pl.pallas_call(
    _ab_body,
    grid=(NBLK,),
    in_specs=[
        pl.BlockSpec((BN, H), lambda i: (i, 0)),
        pl.BlockSpec((2, H, H), lambda i: (0, 0, 0)),
    ],
    out_specs=[
        pl.BlockSpec((BN, H), lambda i: (i, 0)),
        pl.BlockSpec((BN, H), lambda i: (i, 0)),
    ],
    out_shape=[
        jax.ShapeDtypeStruct((NPAD, H), F32),
        jax.ShapeDtypeStruct((NPAD, H), F32),
    ],
)


def _add_into(ab, bb):
    def body(r):
        for c_ in range(H // 16):
            sl = (r, pl.ds(c_ * 16, 16))
            ab[sl] = ab[sl] + bb[sl]

    plsc.parallel_loop(0, SLOT, 1, unroll=4)(body)


@functools.cache
def _build_gather(ne):
    epw = ne // NW
    gout = epw // SLOT

    def body(a_hbm, b_hbm, src_hbm, dst_hbm, so_hbm,
             sidx, didx, ab0, bb0, ab1, bb1, sema, semw0, semw1):
        cid = lax.axis_index("c")
        sid = lax.axis_index("s")
        wid = sid * 2 + cid
        pltpu.sync_copy(src_hbm.at[pl.ds(wid * epw, epw)], sidx)
        pltpu.sync_copy(dst_hbm.at[pl.ds(wid * epw, epw)], didx)

        def issue_gathers(o, ab, bb):
            for j in range(GG):
                pltpu.async_copy(
                    a_hbm.at[sidx.at[pl.ds((o * GG + j) * GC, GC)]],
                    ab.at[pl.ds(j * GC, GC)], sema)
                pltpu.async_copy(
                    b_hbm.at[didx.at[pl.ds((o * GG + j) * GC, GC)]],
                    bb.at[pl.ds(j * GC, GC)], sema)

        def drain_gathers(ab, bb):
            pltpu.make_async_copy(a_hbm.at[pl.ds(0, SLOT)], ab, sema).wait()
            pltpu.make_async_copy(b_hbm.at[pl.ds(0, SLOT)], bb, sema).wait()

        def issue_write(o, ab, semw):
            base = wid * epw + o * SLOT
            pltpu.async_copy(ab, so_hbm.at[pl.ds(base, SLOT)], semw)

        def drain_write(ab, semw):
            pltpu.make_async_copy(ab, so_hbm.at[pl.ds(0, SLOT)], semw).wait()

        issue_gathers(0, ab0, bb0)
        drain_gathers(ab0, bb0)
        issue_gathers(1, ab1, bb1)
        _add_into(ab0, bb0)
        issue_write(0, ab0, semw0)

        def pair(t, carry):
            o1 = 1 + 2 * t
            drain_gathers(ab1, bb1)
            drain_write(ab0, semw0)
            issue_gathers(o1 + 1, ab0, bb0)
            _add_into(ab1, bb1)
            issue_write(o1, ab1, semw1)
            drain_gathers(ab0, bb0)
            drain_write(ab1, semw1)
            issue_gathers(o1 + 2, ab1, bb1)
            _add_into(ab0, bb0)
            issue_write(o1 + 1, ab0, semw0)
            return carry

        if gout % 2 == 0:
            lax.fori_loop(0, (gout - 2) // 2, pair, 0)
            drain_gathers(ab1, bb1)
            _add_into(ab1, bb1)
            issue_write(gout - 1, ab1, semw1)
        else:
            lax.fori_loop(0, (gout - 3) // 2, pair, 0)
            drain_gathers(ab1, bb1)
            drain_write(ab0, semw0)
            issue_gathers(gout - 1, ab0, bb0)
            _add_into(ab1, bb1)
            issue_write(gout - 2, ab1, semw1)
            drain_gathers(ab0, bb0)
            _add_into(ab0, bb0)
            issue_write(gout - 1, ab0, semw0)
        drain_write(ab0, semw0)
        drain_write(ab1, semw1)

    return functools.partial(
        pl.kernel,
        out_type=jax.ShapeDtypeStruct((ne, H), F32),
        mesh=plsc.VectorSubcoreMesh(core_axis_name="c", subcore_axis_name="s"),
        scratch_types=[
            pltpu.VMEM((epw,), jnp.int32),
            pltpu.VMEM((epw,), jnp.int32),
            pltpu.VMEM((SLOT, H), F32),
            pltpu.VMEM((SLOT, H), F32),
            pltpu.VMEM((SLOT, H), F32),
            pltpu.VMEM((SLOT, H), F32),
            pltpu.SemaphoreType.DMA,
            pltpu.SemaphoreType.DMA,
            pltpu.SemaphoreType.DMA,
        ],
    )(body)


def _msg_body(s_ref, e_ref, w1e_ref, b1_ref, w2_ref, b2_ref,
              swt_ref, sb_ref, m_ref):
    t = (s_ref[...]
         + jnp.dot(e_ref[...], w1e_ref[...], preferred_element_type=F32)
         + b1_ref[...])
    t = _silu(t)
    mm = _silu(jnp.dot(t, w2_ref[...], preferred_element_type=F32) + b2_ref[...])
    gate = jax.nn.sigmoid(
        jnp.sum(mm * swt_ref[...], axis=1, keepdims=True) + sb_ref[...])
    m_ref[...] = mm * gate


def _make_msg(ne):
    return pl.pallas_call(
        _msg_body,
        grid=(ne // BE,),
        in_specs=[
            pl.BlockSpec((BE, H), lambda i: (i, 0)),
            pl.BlockSpec((BE, H), lambda i: (i, 0)),
            pl.BlockSpec((H, H), lambda i: (0, 0)),
            pl.BlockSpec((1, H), lambda i: (0, 0)),
            pl.BlockSpec((H, H), lambda i: (0, 0)),
            pl.BlockSpec((1, H), lambda i: (0, 0)),
            pl.BlockSpec((1, H), lambda i: (0, 0)),
            pl.BlockSpec((1, 1), lambda i: (0, 0)),
        ],
        out_specs=pl.BlockSpec((BE, H), lambda i: (i, 0)),
        out_shape=jax.ShapeDtypeStruct((ne, H), F32),
    )


_msg = _make_msg(EH)


@functools.cache
def _build_scatter(ne):
    epw = ne // NW
    nch = epw // SCC
    assert nch % 2 == 1

    def body(m_hbm, dst_hbm, z_hbm, out_hbm,
             didx, mb0, mb1, zbuf, shared, semz, semm0, semm1):
        cid = lax.axis_index("c")
        sid = lax.axis_index("s")
        wid = sid * 2 + cid
        pltpu.sync_copy(z_hbm, zbuf)
        for k in range(NZC):
            pltpu.async_copy(zbuf,
                             shared.at[pl.ds(sid * NROWS_T + k * ZR, ZR)],
                             semz)
        pltpu.sync_copy(dst_hbm.at[wid], didx)
        for k in range(NZC):
            pltpu.make_async_copy(zbuf, shared.at[pl.ds(0, ZR)], semz).wait()
        plsc.subcore_barrier()

        ebase = wid * epw

        def load(k, mb, semm):
            pltpu.async_copy(m_hbm.at[pl.ds(ebase + k * SCC, SCC)], mb, semm)

        def drain(mb, semm):
            pltpu.make_async_copy(m_hbm.at[pl.ds(0, SCC)], mb, semm).wait()

        load(0, mb0, semm0)

        def pair(t, carry):
            k0 = 2 * t
            drain(mb0, semm0)
            load(k0 + 1, mb1, semm1)
            pltpu.sync_copy(mb0, shared.at[didx.at[k0]], add=True)
            drain(mb1, semm1)
            load(k0 + 2, mb0, semm0)
            pltpu.sync_copy(mb1, shared.at[didx.at[k0 + 1]], add=True)
            return carry

        lax.fori_loop(0, (nch - 1) // 2, pair, 0)
        drain(mb0, semm0)
        pltpu.sync_copy(mb0, shared.at[didx.at[nch - 1]], add=True)
        plsc.subcore_barrier()
        for k in range(NZC):
            r0 = sid * NROWS_T + k * ZR
            pltpu.sync_copy(shared.at[pl.ds(r0, ZR)], zbuf)
            pltpu.sync_copy(zbuf, out_hbm.at[cid, pl.ds(r0, ZR)])

    return functools.partial(
        pl.kernel,
        out_type=jax.ShapeDtypeStruct((2, NPAD, H), F32),
        mesh=plsc.VectorSubcoreMesh(core_axis_name="c", subcore_axis_name="s"),
        scratch_types=[
            pltpu.VMEM((nch, SCC), jnp.int32),
            pltpu.VMEM((SCC, H), F32),
            pltpu.VMEM((SCC, H), F32),
            pltpu.VMEM((ZR, H), F32),
            pltpu.VMEM_SHARED((NPAD, H), F32),
            pltpu.SemaphoreType.DMA,
            pltpu.SemaphoreType.DMA,
            pltpu.SemaphoreType.DMA,
        ],
    )(body)


def _make_upd(with_ab):
    def body(p_ref, q_ref, h_ref, w_ref, b_ref, *rest):
        if with_ab:
            wab_ref, h_out, a_out, b_out = rest
        else:
            (h_out,) = rest
        hb = h_ref[...]
        u = p_ref[0] + p_ref[1] + q_ref[0] + q_ref[1] + hb
        hn = _silu(jnp.dot(u, w_ref[0], preferred_element_type=F32) + b_ref[0])
        hn = jnp.dot(hn, w_ref[1], preferred_element_type=F32) + b_ref[1]
        hnew = hn + hb
        h_out[...] = hnew
        if with_ab:
            a_out[...] = jnp.dot(hnew, wab_ref[0], preferred_element_type=F32)
            b_out[...] = jnp.dot(hnew, wab_ref[1], preferred_element_type=F32)

    in_specs = [
        pl.BlockSpec((2, BN, H), lambda i: (0, i, 0)),
        pl.BlockSpec((2, BN, H), lambda i: (0, i, 0)),
        pl.BlockSpec((BN, H), lambda i: (i, 0)),
        pl.BlockSpec((2, H, H), lambda i: (0, 0, 0)),
        pl.BlockSpec((2, 1, H), lambda i: (0, 0, 0)),
    ]
    out_specs = [pl.BlockSpec((BN, H), lambda i: (i, 0))]
    out_shape = [jax.ShapeDtypeStruct((NPAD, H), F32)]
    if with_ab:
        in_specs.append(pl.BlockSpec((2, H, H), lambda i: (0, 0, 0)))
        out_specs += [pl.BlockSpec((BN, H), lambda i: (i, 0))] * 2
        out_shape += [jax.ShapeDtypeStruct((NPAD, H), F32)] * 2
    return pl.pallas_call(
        body, grid=(NBLK,), in_specs=in_specs,
        out_specs=out_specs, out_shape=out_shape,
    )


_upd_ab = _make_upd(True)
_upd = _make_upd(False)


def _read_body(h_ref, ow_ref, ob_ref, rw1_ref, rb1_ref, rw2t_ref, rb2_ref,
               out_ref, acc_ref):
    i = pl.program_id(0)

    @pl.when(i == 0)
    def _():
        acc_ref[...] = jnp.zeros_like(acc_ref)

    hb = h_ref[...]
    ho = _silu(jnp.dot(hb, ow_ref[0], preferred_element_type=F32) + ob_ref[0])
    ho = jnp.dot(ho, ow_ref[1], preferred_element_type=F32) + ob_ref[1]
    rows = lax.broadcasted_iota(jnp.int32, (BN, H), 0) + i * BN
    ho = jnp.where(rows < N, ho, 0.0)
    acc_ref[...] += jnp.sum(ho, axis=0, keepdims=True)

    @pl.when(i == NBLK - 1)
    def _():
        s = acc_ref[...]
        cat = jnp.concatenate([s, s / N], axis=1)
        r = _silu(jnp.dot(cat, rw1_ref[...], preferred_element_type=F32)
                  + rb1_ref[...])
        out_ref[...] = (jnp.sum(r * rw2t_ref[...], axis=1, keepdims=True)
                        + rb2_ref[...])


_readout = pl.pallas_call(
    _read_body,
    grid=(NBLK,),
    in_specs=[
        pl.BlockSpec((BN, H), lambda i: (i, 0)),
        pl.BlockSpec((2, H, H), lambda i: (0, 0, 0)),
        pl.BlockSpec((2, 1, H), lambda i: (0, 0, 0)),
        pl.BlockSpec((2 * H, H), lambda i: (0, 0)),
        pl.BlockSpec((1, H), lambda i: (0, 0)),
        pl.BlockSpec((1, H), lambda i: (0, 0)),
        pl.BlockSpec((1, 1), lambda i: (0, 0)),
    ],
    out_specs=pl.BlockSpec((1, 1), lambda i: (0, 0)),
    out_shape=jax.ShapeDtypeStruct((1, TARGET), F32),
    scratch_shapes=[pltpu.VMEM((1, H), F32)],
    compiler_params=pltpu.CompilerParams(
        dimension_semantics=("arbitrary",)),
)


def kernel(node_feat, edge_index, edge_attr, params):
    p = params
    nf = jnp.pad(node_feat, ((0, NPAD - N), (0, 16 - NFEAT)))
    ea = jnp.pad(edge_attr, ((0, 0), (0, 8 - EFEAT)))
    src = edge_index[0]
    dst = edge_index[1]
    src_h = (src[:EH], src[EH:])
    dst_h = (dst[:EH], dst[EH:])
    nch_h = EH // NW // SCC
    dst3_h = tuple(d.reshape(NW, nch_h, SCC) for d in dst_h)
    bond = jnp.pad(p["bond_emb"], ((0, 0), (0, AVOCAB - BVOCAB), (0, 0)))

    h = _enc_node(nf, p["atom_emb"])
    e_h = (_enc_edge(ea[:EH], bond), _enc_edge(ea[EH:], bond))

    msg_W1 = p["msg_W1"]
    wab = jnp.stack([msg_W1[0, :H], msg_W1[0, H:2 * H]])
    A, B = _ab(h, wab)
    zrows = jnp.zeros((ZR, H), F32)

    for l in range(L):
        def msg_l(s_sum, e_half, l=l):
            return _msg(s_sum, e_half,
                        msg_W1[l, 2 * H:],
                        p["msg_b1"][l][None, :],
                        p["msg_W2"][l],
                        p["msg_b2"][l][None, :],
                        p["soft_W"][l].T,
                        p["soft_b"][l][None, :])

        s1 = _build_gather(EH)(A, B, src_h[0], dst_h[0])
        m1 = msg_l(s1, e_h[0])
        s2 = _build_gather(EH)(A, B, src_h[1], dst_h[1])
        p1 = _build_scatter(EH)(m1, dst3_h[0], zrows)
        m2 = msg_l(s2, e_h[1])
        p2 = _build_scatter(EH)(m2, dst3_h[1], zrows)
        uw = jnp.stack([p["upd_W1"][l], p["upd_W2"][l]])
        ub = jnp.stack([p["upd_b1"][l], p["upd_b2"][l]])[:, None, :]
        if l < L - 1:
            wabn = jnp.stack([msg_W1[l + 1, :H], msg_W1[l + 1, H:2 * H]])
            h, A, B = _upd_ab(p1, p2, h, uw, ub, wabn)
        else:
            (h,) = _upd(p1, p2, h, uw, ub)

    ow = jnp.stack([p["on_W1"], p["on_W2"]])
    ob = jnp.stack([p["on_b1"], p["on_b2"]])[:, None, :]
    out = _readout(h, ow, ob,
                   p["ro_W1"],
                   p["ro_b1"][None, :],
                   p["ro_W2"].T,
                   p["ro_b2"][None, :])
    return out


# speedup vs baseline: 3.5938x; 1.0036x over previous
import functools

import jax
import jax.numpy as jnp
from jax import lax
from jax.experimental import pallas as pl
from jax.experimental.pallas import tpu as pltpu
from jax.experimental.pallas import tpu_sc as plsc

N = 10000
E = 320000
H = 128
L = 4
NFEAT = 9
EFEAT = 3
AVOCAB = 128
BVOCAB = 8
TARGET = 1

NPAD = 10240
BN = 1024
BE = 2000
NBLK = NPAD // BN
EBLK = E // BE

NW = 32
EH = E // 2
GC = 40
GG = 5
SLOT = GC * GG
SCC = 40
NROWS_T = NPAD // 16
ZR = 64
NZC = NROWS_T // ZR

F32 = jnp.float32


def _silu(x):
    return x * jax.nn.sigmoid(x)


def _make_encoder(nrows, nfeat, ncols, block):
    def body(f_ref, emb_ref, out_ref):
        feats = f_ref[...]
        iota = lax.broadcasted_iota(jnp.int32, (block, AVOCAB), 1)
        acc = jnp.zeros((block, H), F32)
        for i in range(nfeat):
            oh = (feats[:, i][:, None] == iota).astype(F32)
            acc = acc + jnp.dot(oh, emb_ref[i], preferred_element_type=F32)
        out_ref[...] = acc

    return pl.pallas_call(
        body,
        grid=(nrows // block,),
        in_specs=[
            pl.BlockSpec((block, ncols), lambda i: (i, 0)),
            pl.BlockSpec((nfeat, AVOCAB, H), lambda i: (0, 0, 0)),
        ],
        out_specs=pl.BlockSpec((block, H), lambda i: (i, 0)),
        out_shape=jax.ShapeDtypeStruct((nrows, H), F32),
    )


_enc_node = _make_encoder(NPAD, NFEAT, 16, BN)
_enc_edge = _make_encoder(EH, EFEAT, 8, BE)


def _ab_body(h_ref, w_ref, a_ref, b_ref):
    hb = h_ref[...]
    a_ref[...] = jnp.dot(hb, w_ref[0], preferred_element_type=F32)
    b_ref[...] = jnp.dot(hb, w_ref[1], preferred_element_type=F32)


_ab = pl.pallas_call(
    _ab_body,
    grid=(NBLK,),
    in_specs=[
        pl.BlockSpec((BN, H), lambda i: (i, 0)),
        pl.BlockSpec((2, H, H), lambda i: (0, 0, 0)),
    ],
    out_specs=[
        pl.BlockSpec((BN, H), lambda i: (i, 0)),
        pl.BlockSpec((BN, H), lambda i: (i, 0)),
    ],
    out_shape=[
        jax.ShapeDtypeStruct((NPAD, H), F32),
        jax.ShapeDtypeStruct((NPAD, H), F32),
    ],
)


def _add_into(ab, bb):
    def body(r):
        for c_ in range(H // 16):
            sl = (r, pl.ds(c_ * 16, 16))
            ab[sl] = ab[sl] + bb[sl]

    plsc.parallel_loop(0, SLOT, 1, unroll=4)(body)


@functools.cache
def _build_gather(ne):
    epw = ne // NW
    gout = epw // SLOT

    def body(a_hbm, b_hbm, src_hbm, dst_hbm, so_hbm,
             sidx, didx, ab0, bb0, ab1, bb1, sema, semw0, semw1):
        cid = lax.axis_index("c")
        sid = lax.axis_index("s")
        wid = sid * 2 + cid
        pltpu.sync_copy(src_hbm.at[pl.ds(wid * epw, epw)], sidx)
        pltpu.sync_copy(dst_hbm.at[pl.ds(wid * epw, epw)], didx)

        def issue_gathers(o, ab, bb):
            for j in range(GG):
                pltpu.async_copy(
                    a_hbm.at[sidx.at[pl.ds((o * GG + j) * GC, GC)]],
                    ab.at[pl.ds(j * GC, GC)], sema)
                pltpu.async_copy(
                    b_hbm.at[didx.at[pl.ds((o * GG + j) * GC, GC)]],
                    bb.at[pl.ds(j * GC, GC)], sema)

        def drain_gathers(ab, bb):
            pltpu.make_async_copy(a_hbm.at[pl.ds(0, SLOT)], ab, sema).wait()
            pltpu.make_async_copy(b_hbm.at[pl.ds(0, SLOT)], bb, sema).wait()

        def issue_write(o, ab, semw):
            base = wid * epw + o * SLOT
            pltpu.async_copy(ab, so_hbm.at[pl.ds(base, SLOT)], semw)

        def drain_write(ab, semw):
            pltpu.make_async_copy(ab, so_hbm.at[pl.ds(0, SLOT)], semw).wait()

        issue_gathers(0, ab0, bb0)
        drain_gathers(ab0, bb0)
        issue_gathers(1, ab1, bb1)
        _add_into(ab0, bb0)
        issue_write(0, ab0, semw0)

        def pair(t, carry):
            o1 = 1 + 2 * t
            drain_gathers(ab1, bb1)
            drain_write(ab0, semw0)
            issue_gathers(o1 + 1, ab0, bb0)
            _add_into(ab1, bb1)
            issue_write(o1, ab1, semw1)
            drain_gathers(ab0, bb0)
            drain_write(ab1, semw1)
            issue_gathers(o1 + 2, ab1, bb1)
            _add_into(ab0, bb0)
            issue_write(o1 + 1, ab0, semw0)
            return carry

        if gout % 2 == 0:
            lax.fori_loop(0, (gout - 2) // 2, pair, 0)
            drain_gathers(ab1, bb1)
            _add_into(ab1, bb1)
            issue_write(gout - 1, ab1, semw1)
        else:
            lax.fori_loop(0, (gout - 3) // 2, pair, 0)
            drain_gathers(ab1, bb1)
            drain_write(ab0, semw0)
            issue_gathers(gout - 1, ab0, bb0)
            _add_into(ab1, bb1)
            issue_write(gout - 2, ab1, semw1)
            drain_gathers(ab0, bb0)
            _add_into(ab0, bb0)
            issue_write(gout - 1, ab0, semw0)
        drain_write(ab0, semw0)
        drain_write(ab1, semw1)

    return functools.partial(
        pl.kernel,
        out_type=jax.ShapeDtypeStruct((ne, H), F32),
        mesh=plsc.VectorSubcoreMesh(core_axis_name="c", subcore_axis_name="s"),
        scratch_types=[
            pltpu.VMEM((epw,), jnp.int32),
            pltpu.VMEM((epw,), jnp.int32),
            pltpu.VMEM((SLOT, H), F32),
            pltpu.VMEM((SLOT, H), F32),
            pltpu.VMEM((SLOT, H), F32),
            pltpu.VMEM((SLOT, H), F32),
            pltpu.SemaphoreType.DMA,
            pltpu.SemaphoreType.DMA,
            pltpu.SemaphoreType.DMA,
        ],
    )(body)


def _msg_body(s_ref, e_ref, w1e_ref, b1_ref, w2_ref, b2_ref,
              swt_ref, sb_ref, m_ref):
    t = (s_ref[...]
         + jnp.dot(e_ref[...], w1e_ref[...], preferred_element_type=F32)
         + b1_ref[...])
    t = _silu(t)
    mm = _silu(jnp.dot(t, w2_ref[...], preferred_element_type=F32) + b2_ref[...])
    gate = jax.nn.sigmoid(
        jnp.sum(mm * swt_ref[...], axis=1, keepdims=True) + sb_ref[...])
    m_ref[...] = mm * gate


def _make_msg(ne):
    return pl.pallas_call(
        _msg_body,
        grid=(ne // BE,),
        in_specs=[
            pl.BlockSpec((BE, H), lambda i: (i, 0)),
            pl.BlockSpec((BE, H), lambda i: (i, 0)),
            pl.BlockSpec((H, H), lambda i: (0, 0)),
            pl.BlockSpec((1, H), lambda i: (0, 0)),
            pl.BlockSpec((H, H), lambda i: (0, 0)),
            pl.BlockSpec((1, H), lambda i: (0, 0)),
            pl.BlockSpec((1, H), lambda i: (0, 0)),
            pl.BlockSpec((1, 1), lambda i: (0, 0)),
        ],
        out_specs=pl.BlockSpec((BE, H), lambda i: (i, 0)),
        out_shape=jax.ShapeDtypeStruct((ne, H), F32),
    )


_msg = _make_msg(EH)


@functools.cache
def _build_scatter(ne):
    epw = ne // NW
    nch = epw // SCC
    assert nch % 2 == 1

    def body(m_hbm, dst_hbm, z_hbm, out_hbm,
             didx, mb0, mb1, zbuf, zbuf2, shared, semz, semm0, semm1):
        cid = lax.axis_index("c")
        sid = lax.axis_index("s")
        wid = sid * 2 + cid
        ebase = wid * epw

        def load(k, mb, semm):
            pltpu.async_copy(m_hbm.at[pl.ds(ebase + k * SCC, SCC)], mb, semm)

        def drain(mb, semm):
            pltpu.make_async_copy(m_hbm.at[pl.ds(0, SCC)], mb, semm).wait()

        pltpu.sync_copy(z_hbm, zbuf)
        for k in range(NZC):
            pltpu.async_copy(zbuf,
                             shared.at[pl.ds(sid * NROWS_T + k * ZR, ZR)],
                             semz)
        pltpu.sync_copy(dst_hbm.at[wid], didx)
        load(0, mb0, semm0)
        for k in range(NZC):
            pltpu.make_async_copy(zbuf, shared.at[pl.ds(0, ZR)], semz).wait()
        plsc.subcore_barrier()

        def pair(t, carry):
            k0 = 2 * t
            drain(mb0, semm0)
            load(k0 + 1, mb1, semm1)
            pltpu.sync_copy(mb0, shared.at[didx.at[k0]], add=True)
            drain(mb1, semm1)
            load(k0 + 2, mb0, semm0)
            pltpu.sync_copy(mb1, shared.at[didx.at[k0 + 1]], add=True)
            return carry

        lax.fori_loop(0, (nch - 1) // 2, pair, 0)
        drain(mb0, semm0)
        pltpu.sync_copy(mb0, shared.at[didx.at[nch - 1]], add=True)
        plsc.subcore_barrier()
        zbs = (zbuf, zbuf2)
        hs = []
        for k in range(NZC):
            if k >= 2:
                hs[k - 2].wait()
            r0 = sid * NROWS_T + k * ZR
            pltpu.sync_copy(shared.at[pl.ds(r0, ZR)], zbs[k % 2])
            hs.append(pltpu.async_copy(zbs[k % 2],
                                       out_hbm.at[cid, pl.ds(r0, ZR)], semz))
        hs[NZC - 2].wait()
        hs[NZC - 1].wait()

    return functools.partial(
        pl.kernel,
        out_type=jax.ShapeDtypeStruct((2, NPAD, H), F32),
        mesh=plsc.VectorSubcoreMesh(core_axis_name="c", subcore_axis_name="s"),
        scratch_types=[
            pltpu.VMEM((nch, SCC), jnp.int32),
            pltpu.VMEM((SCC, H), F32),
            pltpu.VMEM((SCC, H), F32),
            pltpu.VMEM((ZR, H), F32),
            pltpu.VMEM((ZR, H), F32),
            pltpu.VMEM_SHARED((NPAD, H), F32),
            pltpu.SemaphoreType.DMA,
            pltpu.SemaphoreType.DMA,
            pltpu.SemaphoreType.DMA,
        ],
    )(body)


def _make_upd(with_ab):
    def body(p_ref, q_ref, h_ref, w_ref, b_ref, *rest):
        if with_ab:
            wab_ref, h_out, a_out, b_out = rest
        else:
            (h_out,) = rest
        hb = h_ref[...]
        u = p_ref[0] + p_ref[1] + q_ref[0] + q_ref[1] + hb
        hn = _silu(jnp.dot(u, w_ref[0], preferred_element_type=F32) + b_ref[0])
        hn = jnp.dot(hn, w_ref[1], preferred_element_type=F32) + b_ref[1]
        hnew = hn + hb
        h_out[...] = hnew
        if with_ab:
            a_out[...] = jnp.dot(hnew, wab_ref[0], preferred_element_type=F32)
            b_out[...] = jnp.dot(hnew, wab_ref[1], preferred_element_type=F32)

    in_specs = [
        pl.BlockSpec((2, BN, H), lambda i: (0, i, 0)),
        pl.BlockSpec((2, BN, H), lambda i: (0, i, 0)),
        pl.BlockSpec((BN, H), lambda i: (i, 0)),
        pl.BlockSpec((2, H, H), lambda i: (0, 0, 0)),
        pl.BlockSpec((2, 1, H), lambda i: (0, 0, 0)),
    ]
    out_specs = [pl.BlockSpec((BN, H), lambda i: (i, 0))]
    out_shape = [jax.ShapeDtypeStruct((NPAD, H), F32)]
    if with_ab:
        in_specs.append(pl.BlockSpec((2, H, H), lambda i: (0, 0, 0)))
        out_specs += [pl.BlockSpec((BN, H), lambda i: (i, 0))] * 2
        out_shape += [jax.ShapeDtypeStruct((NPAD, H), F32)] * 2
    return pl.pallas_call(
        body, grid=(NBLK,), in_specs=in_specs,
        out_specs=out_specs, out_shape=out_shape,
    )


_upd_ab = _make_upd(True)
_upd = _make_upd(False)


def _read_body(h_ref, ow_ref, ob_ref, rw1_ref, rb1_ref, rw2t_ref, rb2_ref,
               out_ref, acc_ref):
    i = pl.program_id(0)

    @pl.when(i == 0)
    def _():
        acc_ref[...] = jnp.zeros_like(acc_ref)

    hb = h_ref[...]
    ho = _silu(jnp.dot(hb, ow_ref[0], preferred_element_type=F32) + ob_ref[0])
    ho = jnp.dot(ho, ow_ref[1], preferred_element_type=F32) + ob_ref[1]
    rows = lax.broadcasted_iota(jnp.int32, (BN, H), 0) + i * BN
    ho = jnp.where(rows < N, ho, 0.0)
    acc_ref[...] += jnp.sum(ho, axis=0, keepdims=True)

    @pl.when(i == NBLK - 1)
    def _():
        s = acc_ref[...]
        cat = jnp.concatenate([s, s / N], axis=1)
        r = _silu(jnp.dot(cat, rw1_ref[...], preferred_element_type=F32)
                  + rb1_ref[...])
        out_ref[...] = (jnp.sum(r * rw2t_ref[...], axis=1, keepdims=True)
                        + rb2_ref[...])


_readout = pl.pallas_call(
    _read_body,
    grid=(NBLK,),
    in_specs=[
        pl.BlockSpec((BN, H), lambda i: (i, 0)),
        pl.BlockSpec((2, H, H), lambda i: (0, 0, 0)),
        pl.BlockSpec((2, 1, H), lambda i: (0, 0, 0)),
        pl.BlockSpec((2 * H, H), lambda i: (0, 0)),
        pl.BlockSpec((1, H), lambda i: (0, 0)),
        pl.BlockSpec((1, H), lambda i: (0, 0)),
        pl.BlockSpec((1, 1), lambda i: (0, 0)),
    ],
    out_specs=pl.BlockSpec((1, 1), lambda i: (0, 0)),
    out_shape=jax.ShapeDtypeStruct((1, TARGET), F32),
    scratch_shapes=[pltpu.VMEM((1, H), F32)],
    compiler_params=pltpu.CompilerParams(
        dimension_semantics=("arbitrary",)),
)


def kernel(node_feat, edge_index, edge_attr, params):
    p = params
    nf = jnp.pad(node_feat, ((0, NPAD - N), (0, 16 - NFEAT)))
    ea = jnp.pad(edge_attr, ((0, 0), (0, 8 - EFEAT)))
    src = edge_index[0]
    dst = edge_index[1]
    src_h = (src[:EH], src[EH:])
    dst_h = (dst[:EH], dst[EH:])
    nch_h = EH // NW // SCC
    dst3_h = tuple(d.reshape(NW, nch_h, SCC) for d in dst_h)
    bond = jnp.pad(p["bond_emb"], ((0, 0), (0, AVOCAB - BVOCAB), (0, 0)))

    h = _enc_node(nf, p["atom_emb"])
    e_h = (_enc_edge(ea[:EH], bond), _enc_edge(ea[EH:], bond))

    msg_W1 = p["msg_W1"]
    wab = jnp.stack([msg_W1[0, :H], msg_W1[0, H:2 * H]])
    A, B = _ab(h, wab)
    zrows = jnp.zeros((ZR, H), F32)

    for l in range(L):
        def msg_l(s_sum, e_half, l=l):
            return _msg(s_sum, e_half,
                        msg_W1[l, 2 * H:],
                        p["msg_b1"][l][None, :],
                        p["msg_W2"][l],
                        p["msg_b2"][l][None, :],
                        p["soft_W"][l].T,
                        p["soft_b"][l][None, :])

        s1 = _build_gather(EH)(A, B, src_h[0], dst_h[0])
        m1 = msg_l(s1, e_h[0])
        s2 = _build_gather(EH)(A, B, src_h[1], dst_h[1])
        p1 = _build_scatter(EH)(m1, dst3_h[0], zrows)
        m2 = msg_l(s2, e_h[1])
        p2 = _build_scatter(EH)(m2, dst3_h[1], zrows)
        uw = jnp.stack([p["upd_W1"][l], p["upd_W2"][l]])
        ub = jnp.stack([p["upd_b1"][l], p["upd_b2"][l]])[:, None, :]
        if l < L - 1:
            wabn = jnp.stack([msg_W1[l + 1, :H], msg_W1[l + 1, H:2 * H]])
            h, A, B = _upd_ab(p1, p2, h, uw, ub, wabn)
        else:
            (h,) = _upd(p1, p2, h, uw, ub)

    ow = jnp.stack([p["on_W1"], p["on_W2"]])
    ob = jnp.stack([p["on_b1"], p["on_b2"]])[:, None, :]
    out = _readout(h, ow, ob,
                   p["ro_W1"],
                   p["ro_b1"][None, :],
                   p["ro_W2"].T,
                   p["ro_b2"][None, :])
    return out
